# baseline (speedup 1.0000x reference)
import functools

import jax
import jax.numpy as jnp
from jax import lax
from jax.experimental import pallas as pl
from jax.experimental.pallas import tpu as pltpu
from jax.experimental.pallas import tpu_sc as plsc

_SHAPE = (64, 8192)
_TABLE = _SHAPE[1]
_TOTAL = _SHAPE[0] * _SHAPE[1]

_info = plsc.get_sparse_core_info()
_NC, _NS, _L = _info.num_cores, _info.num_subcores, _info.num_lanes
_NW = _NC * _NS
_CHUNK = _TOTAL // _NW
_VECS = _CHUNK // _L


_NSUB = 8
_SUBC = _CHUNK // _NSUB
_SUBV = _SUBC // _L


def _gather_body(x_hbm, idx_hbm, out_hbm, table_v, idx_v, out_v,
                 sem_t, sem_i, sem_o):
    wid = lax.axis_index("s") * _NC + lax.axis_index("c")
    base = wid * _CHUNK
    table_cp = pltpu.async_copy(x_hbm, table_v, sem_t)
    idx_cp = pltpu.async_copy(
        idx_hbm.at[pl.ds(base, _SUBC)], idx_v.at[pl.ds(0, _SUBC)], sem_i)
    table_cp.wait()
    out_cps = []
    for k in range(_NSUB):
        idx_cp.wait()
        if k + 1 < _NSUB:
            off_n = (k + 1) * _SUBC
            idx_cp = pltpu.async_copy(
                idx_hbm.at[pl.ds(base + off_n, _SUBC)],
                idx_v.at[pl.ds(off_n, _SUBC)], sem_i)
        off0 = k * _SUBC

        @plsc.parallel_loop(0, _SUBV, unroll=16)
        def step(i, _off0=off0):
            off = _off0 + i * _L
            iv = idx_v[pl.ds(off, _L)]
            out_v[pl.ds(off, _L)] = plsc.load_gather(table_v, [iv])

        out_cps.append(pltpu.async_copy(
            out_v.at[pl.ds(off0, _SUBC)],
            out_hbm.at[pl.ds(base + off0, _SUBC)], sem_o))
    for cp in out_cps:
        cp.wait()


_gather = pl.kernel(
    _gather_body,
    out_type=jax.ShapeDtypeStruct((_TOTAL,), jnp.float32),
    mesh=plsc.VectorSubcoreMesh(core_axis_name="c", subcore_axis_name="s"),
    scratch_types=[
        pltpu.VMEM((_TABLE,), jnp.float32),
        pltpu.VMEM((_CHUNK,), jnp.int32),
        pltpu.VMEM((_CHUNK,), jnp.float32),
        pltpu.SemaphoreType.DMA,
        pltpu.SemaphoreType.DMA,
        pltpu.SemaphoreType.DMA,
    ],
    compiler_params=pltpu.CompilerParams(needs_layout_passes=False),
)


def kernel(x, idx_tensor):
    idx_flat = idx_tensor.astype(jnp.int32).reshape(_TOTAL)
    return _gather(x, idx_flat).reshape(_SHAPE)


# speedup vs baseline: 1.0993x; 1.0993x over previous
import functools

import jax
import jax.numpy as jnp
from jax import lax
from jax.experimental import pallas as pl
from jax.experimental.pallas import tpu as pltpu
from jax.experimental.pallas import tpu_sc as plsc

_SHAPE = (64, 8192)
_TABLE = _SHAPE[1]
_TOTAL = _SHAPE[0] * _SHAPE[1]

_info = plsc.get_sparse_core_info()
_NC, _NS, _L = _info.num_cores, _info.num_subcores, _info.num_lanes
_NW = _NC * _NS
_CHUNK = _TOTAL // _NW
_VECS = _CHUNK // _L


_NSUB = 4
_SUBC = _CHUNK // _NSUB
_SUBV = _SUBC // _L


def _gather_body(x_hbm, idx_hbm, out_hbm, table_v, idx_v, out_v,
                 sem_t, sem_i, sem_o):
    wid = lax.axis_index("s") * _NC + lax.axis_index("c")
    base = wid * _CHUNK
    table_cp = pltpu.async_copy(x_hbm, table_v, sem_t)
    idx_cp = pltpu.async_copy(
        idx_hbm.at[pl.ds(base, _SUBC)], idx_v.at[pl.ds(0, _SUBC)], sem_i)
    table_cp.wait()
    out_cps = []
    for k in range(_NSUB):
        idx_cp.wait()
        if k + 1 < _NSUB:
            off_n = (k + 1) * _SUBC
            idx_cp = pltpu.async_copy(
                idx_hbm.at[pl.ds(base + off_n, _SUBC)],
                idx_v.at[pl.ds(off_n, _SUBC)], sem_i)
        off0 = k * _SUBC

        @plsc.parallel_loop(0, _SUBV, unroll=16)
        def step(i, _off0=off0):
            off = _off0 + i * _L
            iv = idx_v[pl.ds(off, _L)]
            out_v[pl.ds(off, _L)] = plsc.load_gather(table_v, [iv])

        out_cps.append(pltpu.async_copy(
            out_v.at[pl.ds(off0, _SUBC)],
            out_hbm.at[pl.ds(base + off0, _SUBC)], sem_o))
    for cp in out_cps:
        cp.wait()


_gather = pl.kernel(
    _gather_body,
    out_type=jax.ShapeDtypeStruct((_TOTAL,), jnp.float32),
    mesh=plsc.VectorSubcoreMesh(core_axis_name="c", subcore_axis_name="s"),
    scratch_types=[
        pltpu.VMEM((_TABLE,), jnp.float32),
        pltpu.VMEM((_CHUNK,), jnp.int32),
        pltpu.VMEM((_CHUNK,), jnp.float32),
        pltpu.SemaphoreType.DMA,
        pltpu.SemaphoreType.DMA,
        pltpu.SemaphoreType.DMA,
    ],
    compiler_params=pltpu.CompilerParams(needs_layout_passes=False),
)


def kernel(x, idx_tensor):
    idx_flat = idx_tensor.astype(jnp.int32).reshape(_TOTAL)
    return _gather(x, idx_flat).reshape(_SHAPE)


# speedup vs baseline: 1.1017x; 1.0022x over previous
import functools

import jax
import jax.numpy as jnp
from jax import lax
from jax.experimental import pallas as pl
from jax.experimental.pallas import tpu as pltpu
from jax.experimental.pallas import tpu_sc as plsc

_SHAPE = (64, 8192)
_TABLE = _SHAPE[1]
_TOTAL = _SHAPE[0] * _SHAPE[1]

_info = plsc.get_sparse_core_info()
_NC, _NS, _L = _info.num_cores, _info.num_subcores, _info.num_lanes
_NW = _NC * _NS
_CHUNK = _TOTAL // _NW
_VECS = _CHUNK // _L


_NSUB = 4
_SUBC = _CHUNK // _NSUB
_SUBV = _SUBC // _L


def _gather_body(x_hbm, idx_hbm, out_hbm, table_v, idx_v, out_v,
                 sem_t, sem_i, sem_o):
    wid = lax.axis_index("s") * _NC + lax.axis_index("c")
    base = wid * _CHUNK
    table_cp = pltpu.async_copy(x_hbm, table_v, sem_t)
    idx_cp = pltpu.async_copy(
        idx_hbm.at[pl.ds(base, _SUBC)], idx_v.at[pl.ds(0, _SUBC)], sem_i)
    table_cp.wait()
    out_cps = []
    for k in range(_NSUB):
        idx_cp.wait()
        if k + 1 < _NSUB:
            off_n = (k + 1) * _SUBC
            idx_cp = pltpu.async_copy(
                idx_hbm.at[pl.ds(base + off_n, _SUBC)],
                idx_v.at[pl.ds(off_n, _SUBC)], sem_i)
        off0 = k * _SUBC

        @plsc.parallel_loop(0, _SUBV, unroll=16)
        def step(i, _off0=off0):
            off = _off0 + i * _L
            iv = idx_v[pl.ds(off, _L)]
            out_v[pl.ds(off, _L)] = plsc.load_gather(table_v, [iv])

        out_cps.append(pltpu.async_copy(
            out_v.at[pl.ds(off0, _SUBC)],
            out_hbm.at[pl.ds(base + off0, _SUBC)], sem_o))
    for cp in out_cps:
        cp.wait()


_gather = pl.kernel(
    _gather_body,
    out_type=jax.ShapeDtypeStruct((_TOTAL,), jnp.float32),
    mesh=plsc.VectorSubcoreMesh(core_axis_name="c", subcore_axis_name="s"),
    scratch_types=[
        pltpu.VMEM((_TABLE,), jnp.float32),
        pltpu.VMEM((_CHUNK,), jnp.int32),
        pltpu.VMEM((_CHUNK,), jnp.float32),
        pltpu.SemaphoreType.DMA,
        pltpu.SemaphoreType.DMA,
        pltpu.SemaphoreType.DMA,
    ],
    compiler_params=pltpu.CompilerParams(
        needs_layout_passes=False, skip_device_barrier=True),
)


def kernel(x, idx_tensor):
    idx_flat = idx_tensor.astype(jnp.int32).reshape(_TOTAL)
    return _gather(x, idx_flat).reshape(_SHAPE)


# speedup vs baseline: 1.1622x; 1.0550x over previous
import jax
import jax.numpy as jnp
from jax import lax
from jax.experimental import pallas as pl
from jax.experimental.pallas import tpu as pltpu
from jax.experimental.pallas import tpu_sc as plsc

_SHAPE = (64, 8192)
_TABLE = _SHAPE[1]

_info = plsc.get_sparse_core_info()
_NC, _NS, _L = _info.num_cores, _info.num_subcores, _info.num_lanes
_NW = _NC * _NS
_BR, _BC = 8, 2048
_RG = _SHAPE[0] // _BR
_CG = _SHAPE[1] // _BC
_NSUB = 4
_SCC = _BC // _NSUB
_SUBV = _SCC // _L


def _gather_body(x_hbm, idx_hbm, out_hbm, table_v, idx_v, out_v,
                 sem_t, sem_i, sem_o):
    wid = lax.axis_index("s") * _NC + lax.axis_index("c")
    r0 = (wid // _CG) * _BR
    c0 = (wid % _CG) * _BC
    table_cp = pltpu.async_copy(x_hbm, table_v, sem_t)
    idx_cp = pltpu.async_copy(
        idx_hbm.at[pl.ds(r0, _BR), pl.ds(c0, _SCC)],
        idx_v.at[:, pl.ds(0, _SCC)], sem_i)
    table_cp.wait()
    out_cps = []
    for k in range(_NSUB):
        idx_cp.wait()
        if k + 1 < _NSUB:
            cn = (k + 1) * _SCC
            idx_cp = pltpu.async_copy(
                idx_hbm.at[pl.ds(r0, _BR), pl.ds(c0 + cn, _SCC)],
                idx_v.at[:, pl.ds(cn, _SCC)], sem_i)
        ck = k * _SCC
        for r in range(_BR):

            @plsc.parallel_loop(0, _SUBV, unroll=8)
            def step(i, _r=r, _ck=ck):
                off = _ck + i * _L
                iv = idx_v[_r, pl.ds(off, _L)]
                out_v[_r, pl.ds(off, _L)] = plsc.load_gather(table_v, [iv])

        out_cps.append(pltpu.async_copy(
            out_v.at[:, pl.ds(ck, _SCC)],
            out_hbm.at[pl.ds(r0, _BR), pl.ds(c0 + ck, _SCC)], sem_o))
    for cp in out_cps:
        cp.wait()


_gather = pl.kernel(
    _gather_body,
    out_type=jax.ShapeDtypeStruct(_SHAPE, jnp.float32),
    mesh=plsc.VectorSubcoreMesh(core_axis_name="c", subcore_axis_name="s"),
    scratch_types=[
        pltpu.VMEM((_TABLE,), jnp.float32),
        pltpu.VMEM((_BR, _BC), jnp.int32),
        pltpu.VMEM((_BR, _BC), jnp.float32),
        pltpu.SemaphoreType.DMA,
        pltpu.SemaphoreType.DMA,
        pltpu.SemaphoreType.DMA,
    ],
    compiler_params=pltpu.CompilerParams(
        needs_layout_passes=False, use_tc_tiling_on_sc=True),
)


def kernel(x, idx_tensor):
    return _gather(x, idx_tensor.astype(jnp.int32))


# speedup vs baseline: 1.2579x; 1.0823x over previous
import jax
import jax.numpy as jnp
from jax import lax
from jax.experimental import pallas as pl
from jax.experimental.pallas import tpu as pltpu
from jax.experimental.pallas import tpu_sc as plsc

_SHAPE = (64, 8192)
_TABLE = _SHAPE[1]

_info = plsc.get_sparse_core_info()
_NC, _NS, _L = _info.num_cores, _info.num_subcores, _info.num_lanes
_NW = _NC * _NS
_BR, _BC = 8, 2048
_RG = _SHAPE[0] // _BR
_CG = _SHAPE[1] // _BC
_NSUB = 4
_SCC = _BC // _NSUB
_SUBV = _SCC // _L


def _gather_body(x_hbm, idx_hbm, out_hbm, table_v, idx_v, out_v,
                 sem_t, sem_i, sem_o):
    wid = lax.axis_index("s") * _NC + lax.axis_index("c")
    r0 = (wid // _CG) * _BR
    c0 = (wid % _CG) * _BC
    table_cp = pltpu.async_copy(x_hbm, table_v, sem_t)
    idx_cp = pltpu.async_copy(
        idx_hbm.at[pl.ds(r0, _BR), pl.ds(c0, _SCC)],
        idx_v.at[:, pl.ds(0, _SCC)], sem_i)
    table_cp.wait()
    out_cps = []
    for k in range(_NSUB):
        idx_cp.wait()
        if k + 1 < _NSUB:
            cn = (k + 1) * _SCC
            idx_cp = pltpu.async_copy(
                idx_hbm.at[pl.ds(r0, _BR), pl.ds(c0 + cn, _SCC)],
                idx_v.at[:, pl.ds(cn, _SCC)], sem_i)
        ck = k * _SCC

        @plsc.parallel_loop(0, _BR * _SUBV, unroll=8)
        def step(i, _ck=ck):
            r = i // _SUBV
            off = _ck + (i % _SUBV) * _L
            iv = idx_v[r, pl.ds(off, _L)]
            out_v[r, pl.ds(off, _L)] = plsc.load_gather(table_v, [iv])

        out_cps.append(pltpu.async_copy(
            out_v.at[:, pl.ds(ck, _SCC)],
            out_hbm.at[pl.ds(r0, _BR), pl.ds(c0 + ck, _SCC)], sem_o))
    for cp in out_cps:
        cp.wait()


_gather = pl.kernel(
    _gather_body,
    out_type=jax.ShapeDtypeStruct(_SHAPE, jnp.float32),
    mesh=plsc.VectorSubcoreMesh(core_axis_name="c", subcore_axis_name="s"),
    scratch_types=[
        pltpu.VMEM((_TABLE,), jnp.float32),
        pltpu.VMEM((_BR, _BC), jnp.int32),
        pltpu.VMEM((_BR, _BC), jnp.float32),
        pltpu.SemaphoreType.DMA,
        pltpu.SemaphoreType.DMA,
        pltpu.SemaphoreType.DMA,
    ],
    compiler_params=pltpu.CompilerParams(
        needs_layout_passes=False, use_tc_tiling_on_sc=True),
)


def kernel(x, idx_tensor):
    return _gather(x, idx_tensor.astype(jnp.int32))
